# baseline (speedup 1.0000x reference)
import functools

import jax
import jax.numpy as jnp
from jax import lax
from jax.experimental import pallas as pl
from jax.experimental.pallas import tpu as pltpu
from jax.experimental.pallas import tpu_sc as plsc

_IDX_CHUNK = 128


def _make_gather(V, D, nrows, base, NC, NS):
    NW = NC * NS
    bpw = nrows // NW
    ch = bpw // _IDX_CHUNK
    mesh = plsc.VectorSubcoreMesh(core_axis_name="c", subcore_axis_name="s")

    @functools.partial(
        pl.kernel,
        mesh=mesh,
        out_type=jax.ShapeDtypeStruct((nrows, D), jnp.float32),
        scratch_types=[
            pltpu.VMEM((bpw,), jnp.int32),
            pltpu.VMEM((bpw, D), jnp.float32),
        ]
        + [pltpu.SemaphoreType.DMA] * ch
        + [pltpu.SemaphoreType.DMA],
        compiler_params=pltpu.CompilerParams(use_tc_tiling_on_sc=True),
    )
    def gather_kernel(idx_hbm, table_hbm, out_hbm, idx_v, rows_v, *sems):
        gsems, osem = sems[:ch], sems[ch]
        wid = lax.axis_index("s") * NC + lax.axis_index("c")
        pltpu.sync_copy(idx_hbm.at[pl.ds(base + wid * bpw, bpw)], idx_v)
        copies = [
            pltpu.async_copy(
                table_hbm.at[idx_v.at[pl.ds(j * _IDX_CHUNK, _IDX_CHUNK)]],
                rows_v.at[pl.ds(j * _IDX_CHUNK, _IDX_CHUNK)],
                gsems[j],
            )
            for j in range(ch)
        ]
        out_copies = []
        for j in range(ch):
            copies[j].wait()
            out_copies.append(
                pltpu.async_copy(
                    rows_v.at[pl.ds(j * _IDX_CHUNK, _IDX_CHUNK)],
                    out_hbm.at[pl.ds(wid * bpw + j * _IDX_CHUNK, _IDX_CHUNK)],
                    osem,
                )
            )
        for c in out_copies:
            c.wait()

    return gather_kernel


def _mlp_body(x_ref, pu_ref, eu_ref, w1_ref, b1_ref, w2_ref, b2_ref,
              w3t_ref, b3_ref, wor_ref, bo_ref, out_ref):
    D = x_ref.shape[1]
    u = pu_ref[...] + eu_ref[...]
    w1 = w1_ref[...]
    c1 = jnp.dot(u, w1[:D], preferred_element_type=jnp.float32) + b1_ref[...]
    h = jnp.dot(x_ref[...], w1[D:], preferred_element_type=jnp.float32) + c1
    h = jnp.maximum(h, 0.0)
    h = jnp.dot(h, w2_ref[...], preferred_element_type=jnp.float32) + b2_ref[...]
    h = jnp.maximum(h, 0.0)
    h = lax.dot_general(h, w3t_ref[...], (((1,), (1,)), ((), ())),
                        preferred_element_type=jnp.float32) + b3_ref[...]
    h = jnp.maximum(h, 0.0)
    logits = jnp.dot(h, wor_ref[...], preferred_element_type=jnp.float32) + bo_ref[...]
    r = jax.nn.sigmoid(logits)
    out_ref[...] = r.reshape(out_ref.shape)


def _mlp(x, Pu, Eu, W1, b1, W2, b2, W3t, b3, Wor, bo, tile):
    B, D = x.shape
    grid = B // tile
    vec = lambda n: pl.BlockSpec((n,), lambda i: (0,))
    full = lambda shape: pl.BlockSpec(shape, lambda i: (0, 0))
    return pl.pallas_call(
        _mlp_body,
        grid=(grid,),
        in_specs=[
            pl.BlockSpec((tile, D), lambda i: (i, 0)),
            full(Pu.shape), full(Eu.shape), full(W1.shape), vec(b1.shape[0]),
            full(W2.shape), vec(b2.shape[0]),
            full(W3t.shape), vec(b3.shape[0]),
            full(Wor.shape), vec(bo.shape[0]),
        ],
        out_specs=pl.BlockSpec((tile // 128, 128), lambda i: (i, 0)),
        out_shape=jax.ShapeDtypeStruct((B // 128, 128), jnp.float32),
        compiler_params=pltpu.CompilerParams(
            dimension_semantics=("arbitrary",),
        ),
    )(x, Pu, Eu, W1, b1, W2, b2, W3t, b3, Wor, bo)


def kernel(item_indices, Pu, Eu, Item, W1, b1, W2, b2, W3, b3, Wo, bo):
    B = item_indices.shape[0]
    V, D = Item.shape
    info = plsc.get_sparse_core_info()
    NC, NS = info.num_cores, info.num_subcores
    idx = item_indices.astype(jnp.int32)
    W3t = W3.T
    Wor = Wo
    nsplit = 2
    Bh = B // nsplit
    parts = []
    for s in range(nsplit):
        g = _make_gather(V, D, Bh, s * Bh, NC, NS)(idx, Item)
        parts.append(_mlp(g, Pu, Eu, W1, b1, W2, b2, W3t, b3, Wor, bo,
                          tile=4096))
    out = jnp.concatenate(parts, axis=0)
    return out.reshape(B, 1)


# speedup vs baseline: 1.0528x; 1.0528x over previous
import functools

import jax
import jax.numpy as jnp
from jax import lax
from jax.experimental import pallas as pl
from jax.experimental.pallas import tpu as pltpu
from jax.experimental.pallas import tpu_sc as plsc

_IDX_CHUNK = 128


def _make_gather(V, D, nrows, base, NC, NS):
    NW = NC * NS
    bpw = nrows // NW
    ch = bpw // _IDX_CHUNK
    mesh = plsc.VectorSubcoreMesh(core_axis_name="c", subcore_axis_name="s")

    @functools.partial(
        pl.kernel,
        mesh=mesh,
        out_type=jax.ShapeDtypeStruct((nrows, D), jnp.float32),
        scratch_types=[
            pltpu.VMEM((bpw,), jnp.int32),
            pltpu.VMEM((bpw, D), jnp.float32),
        ]
        + [pltpu.SemaphoreType.DMA] * ch
        + [pltpu.SemaphoreType.DMA],
        compiler_params=pltpu.CompilerParams(use_tc_tiling_on_sc=True),
    )
    def gather_kernel(idx_hbm, table_hbm, out_hbm, idx_v, rows_v, *sems):
        gsems, osem = sems[:ch], sems[ch]
        wid = lax.axis_index("s") * NC + lax.axis_index("c")
        pltpu.sync_copy(idx_hbm.at[pl.ds(base + wid * bpw, bpw)], idx_v)
        copies = [
            pltpu.async_copy(
                table_hbm.at[idx_v.at[pl.ds(j * _IDX_CHUNK, _IDX_CHUNK)]],
                rows_v.at[pl.ds(j * _IDX_CHUNK, _IDX_CHUNK)],
                gsems[j],
            )
            for j in range(ch)
        ]
        out_copies = []
        for j in range(ch):
            copies[j].wait()
            out_copies.append(
                pltpu.async_copy(
                    rows_v.at[pl.ds(j * _IDX_CHUNK, _IDX_CHUNK)],
                    out_hbm.at[pl.ds(wid * bpw + j * _IDX_CHUNK, _IDX_CHUNK)],
                    osem,
                )
            )
        for c in out_copies:
            c.wait()

    return gather_kernel


def _mlp_body(x_ref, pu_ref, eu_ref, w1_ref, b1_ref, w2_ref, b2_ref,
              w3t_ref, b3_ref, wor_ref, bo_ref, out_ref):
    D = x_ref.shape[1]
    u = pu_ref[...] + eu_ref[...]
    w1 = w1_ref[...]
    c1 = jnp.dot(u, w1[:D], preferred_element_type=jnp.float32) + b1_ref[...]
    h = jnp.dot(x_ref[...], w1[D:], preferred_element_type=jnp.float32) + c1
    h = jnp.maximum(h, 0.0)
    h = jnp.dot(h, w2_ref[...], preferred_element_type=jnp.float32) + b2_ref[...]
    h = jnp.maximum(h, 0.0)
    h = lax.dot_general(h, w3t_ref[...], (((1,), (1,)), ((), ())),
                        preferred_element_type=jnp.float32) + b3_ref[...]
    h = jnp.maximum(h, 0.0)
    logits = jnp.dot(h, wor_ref[...], preferred_element_type=jnp.float32) + bo_ref[...]
    r = jax.nn.sigmoid(logits)
    out_ref[...] = r.reshape(out_ref.shape)


def _mlp(x, Pu, Eu, W1, b1, W2, b2, W3t, b3, Wor, bo, tile):
    B, D = x.shape
    grid = B // tile
    vec = lambda n: pl.BlockSpec((n,), lambda i: (0,))
    full = lambda shape: pl.BlockSpec(shape, lambda i: (0, 0))
    return pl.pallas_call(
        _mlp_body,
        grid=(grid,),
        in_specs=[
            pl.BlockSpec((tile, D), lambda i: (i, 0)),
            full(Pu.shape), full(Eu.shape), full(W1.shape), vec(b1.shape[0]),
            full(W2.shape), vec(b2.shape[0]),
            full(W3t.shape), vec(b3.shape[0]),
            full(Wor.shape), vec(bo.shape[0]),
        ],
        out_specs=pl.BlockSpec((tile // 128, 128), lambda i: (i, 0)),
        out_shape=jax.ShapeDtypeStruct((B // 128, 128), jnp.float32),
        compiler_params=pltpu.CompilerParams(
            dimension_semantics=("arbitrary",),
        ),
    )(x, Pu, Eu, W1, b1, W2, b2, W3t, b3, Wor, bo)


def kernel(item_indices, Pu, Eu, Item, W1, b1, W2, b2, W3, b3, Wo, bo):
    B = item_indices.shape[0]
    V, D = Item.shape
    info = plsc.get_sparse_core_info()
    NC, NS = info.num_cores, info.num_subcores
    idx = item_indices.astype(jnp.int32)
    W3t = W3.T
    Wor = Wo
    nsplit = 1
    Bh = B // nsplit
    parts = []
    for s in range(nsplit):
        g = _make_gather(V, D, Bh, s * Bh, NC, NS)(idx, Item)
        parts.append(_mlp(g, Pu, Eu, W1, b1, W2, b2, W3t, b3, Wor, bo,
                          tile=4096))
    out = jnp.concatenate(parts, axis=0)
    return out.reshape(B, 1)


# speedup vs baseline: 1.0576x; 1.0046x over previous
import functools

import jax
import jax.numpy as jnp
from jax import lax
from jax.experimental import pallas as pl
from jax.experimental.pallas import tpu as pltpu
from jax.experimental.pallas import tpu_sc as plsc

_IDX_CHUNK = 128


def _make_gather(V, D, nrows, base, NC, NS):
    NW = NC * NS
    bpw = nrows // NW
    ch = bpw // _IDX_CHUNK
    mesh = plsc.VectorSubcoreMesh(core_axis_name="c", subcore_axis_name="s")

    @functools.partial(
        pl.kernel,
        mesh=mesh,
        out_type=jax.ShapeDtypeStruct((nrows, D), jnp.float32),
        scratch_types=[
            pltpu.VMEM((bpw,), jnp.int32),
            pltpu.VMEM((bpw, D), jnp.float32),
        ]
        + [pltpu.SemaphoreType.DMA] * ch
        + [pltpu.SemaphoreType.DMA],
        compiler_params=pltpu.CompilerParams(use_tc_tiling_on_sc=True),
    )
    def gather_kernel(idx_hbm, table_hbm, out_hbm, idx_v, rows_v, *sems):
        gsems, osem = sems[:ch], sems[ch]
        wid = lax.axis_index("s") * NC + lax.axis_index("c")
        pltpu.sync_copy(idx_hbm.at[pl.ds(base + wid * bpw, bpw)], idx_v)
        copies = [
            pltpu.async_copy(
                table_hbm.at[idx_v.at[pl.ds(j * _IDX_CHUNK, _IDX_CHUNK)]],
                rows_v.at[pl.ds(j * _IDX_CHUNK, _IDX_CHUNK)],
                gsems[j],
            )
            for j in range(ch)
        ]
        out_copies = []
        for j in range(ch):
            copies[j].wait()
            out_copies.append(
                pltpu.async_copy(
                    rows_v.at[pl.ds(j * _IDX_CHUNK, _IDX_CHUNK)],
                    out_hbm.at[pl.ds(wid * bpw + j * _IDX_CHUNK, _IDX_CHUNK)],
                    osem,
                )
            )
        for c in out_copies:
            c.wait()

    return gather_kernel


def _mlp_body(x_ref, pu_ref, eu_ref, w1_ref, b1_ref, w2_ref, b2_ref,
              w3t_ref, b3_ref, wor_ref, bo_ref, out_ref):
    D = x_ref.shape[1]
    u = pu_ref[...] + eu_ref[...]
    w1 = w1_ref[...]
    c1 = jnp.dot(u, w1[:D], preferred_element_type=jnp.float32) + b1_ref[...]
    h = jnp.dot(x_ref[...], w1[D:], preferred_element_type=jnp.float32) + c1
    h = jnp.maximum(h, 0.0)
    h = jnp.dot(h, w2_ref[...], preferred_element_type=jnp.float32) + b2_ref[...]
    h = jnp.maximum(h, 0.0)
    h = lax.dot_general(h, w3t_ref[...], (((1,), (1,)), ((), ())),
                        preferred_element_type=jnp.float32) + b3_ref[...]
    h = jnp.maximum(h, 0.0)
    logits = jnp.dot(h, wor_ref[...], preferred_element_type=jnp.float32) + bo_ref[...]
    r = jax.nn.sigmoid(logits)
    out_ref[...] = r.reshape(out_ref.shape)


def _mlp(x, Pu, Eu, W1, b1, W2, b2, W3t, b3, Wor, bo, tile):
    B, D = x.shape
    grid = B // tile
    vec = lambda n: pl.BlockSpec((n,), lambda i: (0,))
    full = lambda shape: pl.BlockSpec(shape, lambda i: (0, 0))
    return pl.pallas_call(
        _mlp_body,
        grid=(grid,),
        in_specs=[
            pl.BlockSpec((tile, D), lambda i: (i, 0)),
            full(Pu.shape), full(Eu.shape), full(W1.shape), vec(b1.shape[0]),
            full(W2.shape), vec(b2.shape[0]),
            full(W3t.shape), vec(b3.shape[0]),
            full(Wor.shape), vec(bo.shape[0]),
        ],
        out_specs=pl.BlockSpec((tile // 128, 128), lambda i: (i, 0)),
        out_shape=jax.ShapeDtypeStruct((B // 128, 128), jnp.float32),
        compiler_params=pltpu.CompilerParams(
            dimension_semantics=("arbitrary",),
        ),
    )(x, Pu, Eu, W1, b1, W2, b2, W3t, b3, Wor, bo)


def kernel(item_indices, Pu, Eu, Item, W1, b1, W2, b2, W3, b3, Wo, bo):
    B = item_indices.shape[0]
    V, D = Item.shape
    info = plsc.get_sparse_core_info()
    NC, NS = info.num_cores, info.num_subcores
    idx = item_indices.astype(jnp.int32)
    W3t = W3.T
    Wor = Wo
    nsplit = 1
    Bh = B // nsplit
    parts = []
    for s in range(nsplit):
        g = _make_gather(V, D, Bh, s * Bh, NC, NS)(idx, Item)
        parts.append(_mlp(g, Pu, Eu, W1, b1, W2, b2, W3t, b3, Wor, bo,
                          tile=8192))
    out = jnp.concatenate(parts, axis=0)
    return out.reshape(B, 1)


# speedup vs baseline: 1.0823x; 1.0233x over previous
import functools

import jax
import jax.numpy as jnp
from jax import lax
from jax.experimental import pallas as pl
from jax.experimental.pallas import tpu as pltpu
from jax.experimental.pallas import tpu_sc as plsc

_IDX_CHUNK = 128


def _make_gather(V, D, nrows, NC, NS):
    NW = NC * NS
    bpw = nrows // NW
    ch = bpw // _IDX_CHUNK
    mesh = plsc.VectorSubcoreMesh(core_axis_name="c", subcore_axis_name="s")

    @functools.partial(
        pl.kernel,
        mesh=mesh,
        out_type=jax.ShapeDtypeStruct((nrows, D), jnp.float32),
        scratch_types=[
            pltpu.VMEM((bpw,), jnp.int32),
            pltpu.VMEM((bpw, D), jnp.float32),
        ]
        + [pltpu.SemaphoreType.DMA] * ch
        + [pltpu.SemaphoreType.DMA],
        compiler_params=pltpu.CompilerParams(use_tc_tiling_on_sc=True),
    )
    def gather_kernel(idx_hbm, table_hbm, out_hbm, idx_v, rows_v, *sems):
        gsems, osem = sems[:ch], sems[ch]
        wid = lax.axis_index("s") * NC + lax.axis_index("c")
        pltpu.sync_copy(idx_hbm.at[pl.ds(wid * bpw, bpw)], idx_v)
        copies = [
            pltpu.async_copy(
                table_hbm.at[idx_v.at[pl.ds(j * _IDX_CHUNK, _IDX_CHUNK)]],
                rows_v.at[pl.ds(j * _IDX_CHUNK, _IDX_CHUNK)],
                gsems[j],
            )
            for j in range(ch)
        ]
        out_copies = []
        for j in range(ch):
            copies[j].wait()
            out_copies.append(
                pltpu.async_copy(
                    rows_v.at[pl.ds(j * _IDX_CHUNK, _IDX_CHUNK)],
                    out_hbm.at[pl.ds(wid * bpw + j * _IDX_CHUNK, _IDX_CHUNK)],
                    osem,
                )
            )
        for c in out_copies:
            c.wait()

    return gather_kernel


def _mlp_body(x_ref, pu_ref, eu_ref, w1_ref, b1_ref, w2_ref, b2_ref,
              w3t_ref, b3_ref, wor_ref, bo_ref, out_ref):
    D = x_ref.shape[1]
    bf = jnp.bfloat16
    u = pu_ref[...] + eu_ref[...]
    w1 = w1_ref[...]
    c1 = (jnp.dot(u, w1[:D], preferred_element_type=jnp.float32)
          + b1_ref[...]).astype(bf)
    h = jnp.dot(x_ref[...].astype(bf), w1[D:].astype(bf),
                preferred_element_type=jnp.float32).astype(bf)
    h = jnp.maximum(h + c1, bf(0.0))
    h = jnp.dot(h, w2_ref[...].astype(bf),
                preferred_element_type=jnp.float32).astype(bf)
    h = jnp.maximum(h + b2_ref[...].astype(bf), bf(0.0))
    h = lax.dot_general(h, w3t_ref[...].astype(bf), (((1,), (1,)), ((), ())),
                        preferred_element_type=jnp.float32).astype(bf)
    h = jnp.maximum(h + b3_ref[...].astype(bf), bf(0.0))
    logits_t = lax.dot_general(wor_ref[...].astype(bf), h,
                               (((1,), (1,)), ((), ())),
                               preferred_element_type=jnp.float32)
    lg = logits_t.reshape(out_ref.shape) + bo_ref[...]
    out_ref[...] = jax.nn.sigmoid(lg)


def _mlp(x, Pu, Eu, W1, b1, W2, b2, W3t, b3, Wor, bo, tile):
    B, D = x.shape
    grid = B // tile
    vec = lambda n: pl.BlockSpec((n,), lambda i: (0,))
    full = lambda shape: pl.BlockSpec(shape, lambda i: (0, 0))
    return pl.pallas_call(
        _mlp_body,
        grid=(grid,),
        in_specs=[
            pl.BlockSpec((tile, D), lambda i: (i, 0)),
            full(Pu.shape), full(Eu.shape), full(W1.shape), vec(b1.shape[0]),
            full(W2.shape), vec(b2.shape[0]),
            full(W3t.shape), vec(b3.shape[0]),
            full(Wor.shape), vec(bo.shape[0]),
        ],
        out_specs=pl.BlockSpec((tile // 128, 128), lambda i: (i, 0)),
        out_shape=jax.ShapeDtypeStruct((B // 128, 128), jnp.float32),
        compiler_params=pltpu.CompilerParams(
            dimension_semantics=("arbitrary",),
        ),
    )(x, Pu, Eu, W1, b1, W2, b2, W3t, b3, Wor, bo)


def kernel(item_indices, Pu, Eu, Item, W1, b1, W2, b2, W3, b3, Wo, bo):
    B = item_indices.shape[0]
    V, D = Item.shape
    info = plsc.get_sparse_core_info()
    NC, NS = info.num_cores, info.num_subcores
    idx = item_indices.astype(jnp.int32)
    W3t = W3.T
    Wor = Wo.reshape(1, -1)
    g = _make_gather(V, D, B, NC, NS)(idx, Item)
    out = _mlp(g, Pu, Eu, W1, b1, W2, b2, W3t, b3, Wor, bo, tile=8192)
    return out.reshape(B, 1)
